# baseline (speedup 1.0000x reference)
import dataclasses

import jax
import jax.numpy as jnp
import numpy as np
from jax import lax
from jax.experimental import pallas as pl
from jax.experimental.pallas import tpu as pltpu
from jax.experimental.pallas import tpu_sc as plsc

N = 100000
GD = 256
M = 10000
K = 16
MD = 16
L = 256

NW = 32
PER_W = 320
M_PAD = NW * PER_W
CH_N = 8
CH_R = CH_N * K
NCH = PER_W // CH_N
NBUF = 4
LANES = 16

_G = np.arange(16)
_PERM = np.concatenate(
    [np.concatenate([16 * g + _G, 128 + 16 * g + _G]) for g in range(8)])


def _ln_silu(h, g, b):
    m = jnp.mean(h, axis=-1, keepdims=True)
    v = jnp.mean((h - m) ** 2, axis=-1, keepdims=True)
    hn = (h - m) * lax.rsqrt(v + 1e-5) * g + b
    return hn * jax.nn.sigmoid(hn)


def _grid_mlp_body(x_ref, w1_ref, b1_ref, g1_ref, be1_ref, w2_ref, b2_ref,
                   o_ref, o2_ref):
    h = jnp.dot(x_ref[...].astype(jnp.bfloat16), w1_ref[...],
                preferred_element_type=jnp.float32)
    h = _ln_silu(h + b1_ref[...], g1_ref[...], be1_ref[...])
    y = jnp.dot(h.astype(jnp.bfloat16), w2_ref[...],
                preferred_element_type=jnp.float32) + b2_ref[...]
    lo = lax.bitcast_convert_type(
        y[:, :L // 2].astype(jnp.bfloat16).astype(jnp.float32), jnp.int32)
    hi = lax.bitcast_convert_type(
        y[:, L // 2:].astype(jnp.bfloat16).astype(jnp.float32), jnp.int32)
    packed = hi | lax.shift_right_logical(lo, 16)
    o_ref[...] = packed
    o2_ref[...] = packed


def _grid_mlp(x, w1, b1, g1, be1, w2, b2, rb):
    nb = x.shape[0] // rb
    full = pl.BlockSpec((GD, L), lambda i: (0, 0))
    vec = pl.BlockSpec((1, L), lambda i: (0, 0))
    return pl.pallas_call(
        _grid_mlp_body,
        grid=(nb,),
        in_specs=[pl.BlockSpec((rb, GD), lambda i: (i, 0)),
                  full, vec, vec, vec, full, vec],
        out_specs=[pl.BlockSpec((rb, L // 2), lambda i: (i, 0)),
                   pl.BlockSpec((rb, L // 2), lambda i: (i, 0))],
        out_shape=[jax.ShapeDtypeStruct((x.shape[0], L // 2), jnp.int32),
                   jax.ShapeDtypeStruct((x.shape[0], L // 2), jnp.int32)],
    )(x, w1, b1, g1, be1, w2, b2)


def _combine_body(mp_ref, mf_ref, w3a_ref, w3b_ref, b3_ref, g2_ref, be2_ref,
                  w4_ref, b4_ref, o_ref):
    h = jnp.dot(mp_ref[...], w3a_ref[...], preferred_element_type=jnp.float32)
    h = h + jnp.dot(mf_ref[...], w3b_ref[...], preferred_element_type=jnp.float32)
    h = _ln_silu(h + b3_ref[...], g2_ref[...], be2_ref[...])
    o_ref[...] = jnp.dot(h, w4_ref[...], preferred_element_type=jnp.float32) + b4_ref[...]


def _combine_mlp(mp, mf, w3a, w3b, b3, g2, be2, w4, b4, mb):
    nb = mf.shape[0] // mb
    full = pl.BlockSpec((L, L), lambda i: (0, 0))
    vec = pl.BlockSpec((1, L), lambda i: (0, 0))
    return pl.pallas_call(
        _combine_body,
        grid=(nb,),
        in_specs=[pl.BlockSpec((mb, L), lambda i: (i, 0)),
                  pl.BlockSpec((mb, MD), lambda i: (i, 0)),
                  full, pl.BlockSpec((MD, L), lambda i: (0, 0)),
                  vec, vec, vec, full, vec],
        out_specs=pl.BlockSpec((mb, L), lambda i: (i, 0)),
        out_shape=jax.ShapeDtypeStruct((mf.shape[0], L), jnp.float32),
    )(mp, mf, w3a, w3b, b3, g2, be2, w4, b4)


def _sc_body(g_hbm, g2_hbm, idx_hbm, w_hbm, out_hbm, idx_v, w_v,
             rows0, rows1, rows2, rows3, out_v, sem0, sem1, sem2, sem3):
    bufs = (rows0, rows1, rows2, rows3)
    sems = (sem0, sem1, sem2, sem3)
    cid = lax.axis_index("c")
    wid = lax.axis_index("s") * 2 + cid
    node0 = wid * PER_W
    e0 = node0 * K
    pltpu.sync_copy(idx_hbm.at[pl.ds(e0, PER_W * K)], idx_v)
    pltpu.sync_copy(w_hbm.at[pl.ds(e0, PER_W * K)], w_v)

    def fire(c, rows, sem):
        @pl.when(cid == 0)
        def _():
            pltpu.async_copy(g_hbm.at[idx_v.at[pl.ds(c * CH_R, CH_R)]],
                             rows, sem)

        @pl.when(cid == 1)
        def _():
            pltpu.async_copy(g2_hbm.at[idx_v.at[pl.ds(c * CH_R, CH_R)]],
                             rows, sem)

    def compute(c, rows, sem):
        pltpu.make_async_copy(
            g_hbm.at[idx_v.at[pl.ds(0, CH_R)]], rows, sem).wait()
        himask = jnp.full((LANES,), -65536, dtype=jnp.int32)

        @plsc.parallel_loop(0, CH_N, unroll=2)
        def _(mi):
            acc_e = [None] * (L // 32)
            acc_o = [None] * (L // 32)
            for k in range(K):
                r = mi * K + k
                widx = jnp.full((LANES,), c * CH_R + r, dtype=jnp.int32)
                wk = plsc.load_gather(w_v, [widx])
                for gidx in range(L // 32):
                    vi = rows[r, pl.ds(gidx * LANES, LANES)]
                    ev = plsc.bitcast(vi << 16, jnp.float32)
                    od = plsc.bitcast(vi & himask, jnp.float32)
                    te = wk * ev
                    to = wk * od
                    if k == 0:
                        acc_e[gidx] = te
                        acc_o[gidx] = to
                    else:
                        acc_e[gidx] = acc_e[gidx] + te
                        acc_o[gidx] = acc_o[gidx] + to
            for gidx in range(L // 32):
                base = gidx * 32
                out_v[mi, pl.ds(base, LANES)] = acc_e[gidx]
                out_v[mi, pl.ds(base + LANES, LANES)] = acc_o[gidx]
        pltpu.sync_copy(out_v,
                        out_hbm.at[pl.ds(node0 + c * CH_N, CH_N)])

    for b in range(NBUF):
        fire(b, bufs[b], sems[b])

    @pl.loop(0, NCH, step=NBUF)
    def _(c):
        for b in range(NBUF):
            compute(c + b, bufs[b], sems[b])

            def refill(b=b):
                fire(c + b + NBUF, bufs[b], sems[b])

            pl.when(c + b + NBUF < NCH)(refill)


@jax.jit
def _sc_gather_reduce(g, g2, idx_pad, w_pad):
    mesh = plsc.VectorSubcoreMesh(core_axis_name="c", subcore_axis_name="s")
    cp = pltpu.CompilerParams()
    if "needs_layout_passes" in pltpu.CompilerParams.__dataclass_fields__:
        cp = dataclasses.replace(cp, needs_layout_passes=False)
    f = pl.kernel(
        _sc_body,
        out_type=jax.ShapeDtypeStruct((M_PAD, L), jnp.float32),
        mesh=mesh,
        scratch_types=[
            pltpu.VMEM((PER_W * K,), jnp.int32),
            pltpu.VMEM((PER_W * K,), jnp.float32),
            pltpu.VMEM((CH_R, L // 2), jnp.int32),
            pltpu.VMEM((CH_R, L // 2), jnp.int32),
            pltpu.VMEM((CH_R, L // 2), jnp.int32),
            pltpu.VMEM((CH_R, L // 2), jnp.int32),
            pltpu.VMEM((CH_N, L), jnp.float32),
            pltpu.SemaphoreType.DMA,
            pltpu.SemaphoreType.DMA,
            pltpu.SemaphoreType.DMA,
            pltpu.SemaphoreType.DMA,
        ],
        compiler_params=cp,
    )
    return f(g, g2, idx_pad, w_pad)


def kernel(grid_data, mesh_features, g2m_indices, g2m_weights,
           W1, b1, g1, be1, W2, b2, W3, b3, g2, be2, W4, b4):
    x = grid_data.reshape(N, GD)
    table, table2 = _grid_mlp(x, W1.astype(jnp.bfloat16), b1.reshape(1, L),
                              g1.reshape(1, L), be1.reshape(1, L),
                              W2.astype(jnp.bfloat16), b2.reshape(1, L),
                              rb=5000)

    pad = M_PAD * K - M * K
    idx_pad = jnp.concatenate(
        [g2m_indices.reshape(-1).astype(jnp.int32),
         jnp.zeros((pad,), jnp.int32)])
    w_pad = jnp.concatenate(
        [g2m_weights.reshape(-1), jnp.zeros((pad,), jnp.float32)])
    mp = _sc_gather_reduce(table, table2, idx_pad, w_pad)

    w3a = W3[:L][_PERM]
    out = _combine_mlp(mp, mesh_features, w3a, W3[L:], b3.reshape(1, L),
                       g2.reshape(1, L), be2.reshape(1, L), W4,
                       b4.reshape(1, L), mb=2000)
    return out.reshape(1, M, L)


# speedup vs baseline: 1.0063x; 1.0063x over previous
import dataclasses

import jax
import jax.numpy as jnp
import numpy as np
from jax import lax
from jax.experimental import pallas as pl
from jax.experimental.pallas import tpu as pltpu
from jax.experimental.pallas import tpu_sc as plsc

N = 100000
GD = 256
M = 10000
K = 16
MD = 16
L = 256

NW = 32
PER_W = 320
M_PAD = NW * PER_W
CH_N = 16
CH_R = CH_N * K
NCH = PER_W // CH_N
NBUF = 2
LANES = 16

_G = np.arange(16)
_PERM = np.concatenate(
    [np.concatenate([16 * g + _G, 128 + 16 * g + _G]) for g in range(8)])


def _ln_silu(h, g, b):
    m = jnp.mean(h, axis=-1, keepdims=True)
    v = jnp.mean((h - m) ** 2, axis=-1, keepdims=True)
    hn = (h - m) * lax.rsqrt(v + 1e-5) * g + b
    return hn * jax.nn.sigmoid(hn)


def _grid_mlp_body(x_ref, w1_ref, b1_ref, g1_ref, be1_ref, w2_ref, b2_ref,
                   o_ref, o2_ref):
    h = jnp.dot(x_ref[...].astype(jnp.bfloat16), w1_ref[...],
                preferred_element_type=jnp.float32)
    h = _ln_silu(h + b1_ref[...], g1_ref[...], be1_ref[...])
    y = jnp.dot(h.astype(jnp.bfloat16), w2_ref[...],
                preferred_element_type=jnp.float32) + b2_ref[...]
    lo = lax.bitcast_convert_type(
        y[:, :L // 2].astype(jnp.bfloat16).astype(jnp.float32), jnp.int32)
    hi = lax.bitcast_convert_type(
        y[:, L // 2:].astype(jnp.bfloat16).astype(jnp.float32), jnp.int32)
    packed = hi | lax.shift_right_logical(lo, 16)
    o_ref[...] = packed
    o2_ref[...] = packed


def _grid_mlp(x, w1, b1, g1, be1, w2, b2, rb):
    nb = x.shape[0] // rb
    full = pl.BlockSpec((GD, L), lambda i: (0, 0))
    vec = pl.BlockSpec((1, L), lambda i: (0, 0))
    return pl.pallas_call(
        _grid_mlp_body,
        grid=(nb,),
        in_specs=[pl.BlockSpec((rb, GD), lambda i: (i, 0)),
                  full, vec, vec, vec, full, vec],
        out_specs=[pl.BlockSpec((rb, L // 2), lambda i: (i, 0)),
                   pl.BlockSpec((rb, L // 2), lambda i: (i, 0))],
        out_shape=[jax.ShapeDtypeStruct((x.shape[0], L // 2), jnp.int32),
                   jax.ShapeDtypeStruct((x.shape[0], L // 2), jnp.int32)],
    )(x, w1, b1, g1, be1, w2, b2)


def _combine_body(mp_ref, mf_ref, w3a_ref, w3b_ref, b3_ref, g2_ref, be2_ref,
                  w4_ref, b4_ref, o_ref):
    h = jnp.dot(mp_ref[...], w3a_ref[...], preferred_element_type=jnp.float32)
    h = h + jnp.dot(mf_ref[...], w3b_ref[...], preferred_element_type=jnp.float32)
    h = _ln_silu(h + b3_ref[...], g2_ref[...], be2_ref[...])
    o_ref[...] = jnp.dot(h, w4_ref[...], preferred_element_type=jnp.float32) + b4_ref[...]


def _combine_mlp(mp, mf, w3a, w3b, b3, g2, be2, w4, b4, mb):
    nb = mf.shape[0] // mb
    full = pl.BlockSpec((L, L), lambda i: (0, 0))
    vec = pl.BlockSpec((1, L), lambda i: (0, 0))
    return pl.pallas_call(
        _combine_body,
        grid=(nb,),
        in_specs=[pl.BlockSpec((mb, L), lambda i: (i, 0)),
                  pl.BlockSpec((mb, MD), lambda i: (i, 0)),
                  full, pl.BlockSpec((MD, L), lambda i: (0, 0)),
                  vec, vec, vec, full, vec],
        out_specs=pl.BlockSpec((mb, L), lambda i: (i, 0)),
        out_shape=jax.ShapeDtypeStruct((mf.shape[0], L), jnp.float32),
    )(mp, mf, w3a, w3b, b3, g2, be2, w4, b4)


def _sc_body(g_hbm, g2_hbm, idx_hbm, w_hbm, out_hbm, idx_v, w_v,
             rows0, rows1, out_v, sem0, sem1):
    bufs = (rows0, rows1)
    sems = (sem0, sem1)
    cid = lax.axis_index("c")
    wid = lax.axis_index("s") * 2 + cid
    node0 = wid * PER_W
    e0 = node0 * K
    pltpu.sync_copy(idx_hbm.at[pl.ds(e0, PER_W * K)], idx_v)
    pltpu.sync_copy(w_hbm.at[pl.ds(e0, PER_W * K)], w_v)

    def fire(c, rows, sem):
        @pl.when(cid == 0)
        def _():
            pltpu.async_copy(g_hbm.at[idx_v.at[pl.ds(c * CH_R, CH_R)]],
                             rows, sem)

        @pl.when(cid == 1)
        def _():
            pltpu.async_copy(g2_hbm.at[idx_v.at[pl.ds(c * CH_R, CH_R)]],
                             rows, sem)

    def compute(c, rows, sem):
        pltpu.make_async_copy(
            g_hbm.at[idx_v.at[pl.ds(0, CH_R)]], rows, sem).wait()
        himask = jnp.full((LANES,), -65536, dtype=jnp.int32)

        @plsc.parallel_loop(0, CH_N, unroll=2)
        def _(mi):
            acc_e = [None] * (L // 32)
            acc_o = [None] * (L // 32)
            for k in range(K):
                r = mi * K + k
                widx = jnp.full((LANES,), c * CH_R + r, dtype=jnp.int32)
                wk = plsc.load_gather(w_v, [widx])
                for gidx in range(L // 32):
                    vi = rows[r, pl.ds(gidx * LANES, LANES)]
                    ev = plsc.bitcast(vi << 16, jnp.float32)
                    od = plsc.bitcast(vi & himask, jnp.float32)
                    te = wk * ev
                    to = wk * od
                    if k == 0:
                        acc_e[gidx] = te
                        acc_o[gidx] = to
                    else:
                        acc_e[gidx] = acc_e[gidx] + te
                        acc_o[gidx] = acc_o[gidx] + to
            for gidx in range(L // 32):
                base = gidx * 32
                out_v[mi, pl.ds(base, LANES)] = acc_e[gidx]
                out_v[mi, pl.ds(base + LANES, LANES)] = acc_o[gidx]
        pltpu.sync_copy(out_v,
                        out_hbm.at[pl.ds(node0 + c * CH_N, CH_N)])

    for b in range(NBUF):
        fire(b, bufs[b], sems[b])

    @pl.loop(0, NCH, step=NBUF)
    def _(c):
        for b in range(NBUF):
            compute(c + b, bufs[b], sems[b])

            def refill(b=b):
                fire(c + b + NBUF, bufs[b], sems[b])

            pl.when(c + b + NBUF < NCH)(refill)


@jax.jit
def _sc_gather_reduce(g, g2, idx_pad, w_pad):
    mesh = plsc.VectorSubcoreMesh(core_axis_name="c", subcore_axis_name="s")
    cp = pltpu.CompilerParams()
    if "needs_layout_passes" in pltpu.CompilerParams.__dataclass_fields__:
        cp = dataclasses.replace(cp, needs_layout_passes=False)
    f = pl.kernel(
        _sc_body,
        out_type=jax.ShapeDtypeStruct((M_PAD, L), jnp.float32),
        mesh=mesh,
        scratch_types=[
            pltpu.VMEM((PER_W * K,), jnp.int32),
            pltpu.VMEM((PER_W * K,), jnp.float32),
            pltpu.VMEM((CH_R, L // 2), jnp.int32),
            pltpu.VMEM((CH_R, L // 2), jnp.int32),
            pltpu.VMEM((CH_N, L), jnp.float32),
            pltpu.SemaphoreType.DMA,
            pltpu.SemaphoreType.DMA,
        ],
        compiler_params=cp,
    )
    return f(g, g2, idx_pad, w_pad)


def kernel(grid_data, mesh_features, g2m_indices, g2m_weights,
           W1, b1, g1, be1, W2, b2, W3, b3, g2, be2, W4, b4):
    x = grid_data.reshape(N, GD)
    table, table2 = _grid_mlp(x, W1.astype(jnp.bfloat16), b1.reshape(1, L),
                              g1.reshape(1, L), be1.reshape(1, L),
                              W2.astype(jnp.bfloat16), b2.reshape(1, L),
                              rb=5000)

    pad = M_PAD * K - M * K
    idx_pad = jnp.concatenate(
        [g2m_indices.reshape(-1).astype(jnp.int32),
         jnp.zeros((pad,), jnp.int32)])
    w_pad = jnp.concatenate(
        [g2m_weights.reshape(-1), jnp.zeros((pad,), jnp.float32)])
    mp = _sc_gather_reduce(table, table2, idx_pad, w_pad)

    w3a = W3[:L][_PERM]
    out = _combine_mlp(mp, mesh_features, w3a, W3[L:], b3.reshape(1, L),
                       g2.reshape(1, L), be2.reshape(1, L), W4,
                       b4.reshape(1, L), mb=2000)
    return out.reshape(1, M, L)
